# baseline (speedup 1.0000x reference)
import jax
import jax.numpy as jnp
from jax import lax
from jax.experimental import pallas as pl
from jax.experimental.pallas import tpu as pltpu
from jax.experimental.pallas import tpu_sc as plsc

_NUM_FIELDS = 26
_VOCAB = 100000
_D = 32
_BATCH = 16384
_NC, _NS = 2, 16
_NW = _NC * _NS

_TJ = 781
_PAIRS = _NUM_FIELDS * _TJ
_SPW = 636

_TOTAL = _NUM_FIELDS * _BATCH
_PER_W = _TOTAL // _NW
_IDX_W = 128
_IDXROWS_W = _PER_W // _IDX_W
_G = 8
_CHUNK = _G * _IDX_W
_NCHUNK = _IDXROWS_W // _G
_NBUF = 3


def _pair_of(wid, s):
    p = s * _NW + wid
    valid = p < _PAIRS
    pc = jnp.where(valid, p, 0)
    return valid, pc // _TJ, pc % _TJ


def _transpose_body(tt_hbm, edge_hbm, tbl_hbm, tin, tout, isems, osems):
    wid = lax.axis_index("s") * _NC + lax.axis_index("c")
    nb = 2
    iota = lax.iota(jnp.int32, 16)
    dvec0 = iota
    dvec1 = iota + 16

    def fire_read(s, b):
        valid, f, tj = _pair_of(wid, s)

        @pl.when(valid)
        def _():
            for t in range(4):
                pltpu.async_copy(
                    tt_hbm.at[f, pl.ds(t * 8, 8), pl.ds(tj * 128, 128)],
                    tin.at[b].at[pl.ds(t * 8, 8)],
                    isems[b],
                )

    def wait_read(b):
        for t in range(4):
            pltpu.make_async_copy(
                tt_hbm.at[0, pl.ds(0, 8), pl.ds(0, 128)],
                tin.at[b].at[pl.ds(t * 8, 8)],
                isems[b],
            ).wait()

    def wait_write(b):
        pltpu.make_async_copy(
            tout.at[b], tbl_hbm.at[pl.ds(0, _D)], osems[b]
        ).wait()

    def step(s, b, first):
        valid, f, tj = _pair_of(wid, s)

        @pl.when(valid)
        def _():
            wait_read(b)
            if not first:
                wait_write(b)
            bvec = jnp.full((16,), b, jnp.int32)
            for m in range(8):
                dvec = dvec0 if m % 2 == 0 else dvec1
                for r in range(_D):
                    cvec = jnp.full((16,), r * 4 + m // 2, jnp.int32)
                    v = plsc.load_gather(tin, [bvec, dvec, cvec])
                    tout[b, r, pl.ds(m * 16, 16)] = v
            k0 = (f * (_VOCAB // 32) + tj * 4) * 8
            pltpu.async_copy(tout.at[b], tbl_hbm.at[pl.ds(k0, _D)], osems[b])

        fire_read(s + nb, b)

    fire_read(0, 0)
    fire_read(1, 1)
    step(0, 0, True)
    step(1, 1, True)

    def body(t2, carry):
        s0 = nb + 2 * t2
        step(s0, 0, False)
        step(s0 + 1, 1, False)
        return carry

    lax.fori_loop(0, (_SPW - nb) // 2, body, 0)

    for q in range(_SPW - nb, _SPW):
        valid, f, tj = _pair_of(wid, q)

        @pl.when(valid)
        def _():
            wait_write(q % nb)

    @pl.when(wid < _NUM_FIELDS)
    def _():
        f = wid
        pltpu.sync_copy(edge_hbm.at[f], tin.at[0, pl.ds(0, 8), pl.ds(0, 128)])
        k0 = f * (_VOCAB // 4) + (_TJ * 128) // 4
        pltpu.sync_copy(
            tin.at[0, pl.ds(0, 8), pl.ds(0, 128)], tbl_hbm.at[pl.ds(k0, 8)]
        )


def _gather_body(idx_hbm, tbl_hbm, out_hbm, idx_v, rows_v, gsems, wsems):
    wid = lax.axis_index("s") * _NC + lax.axis_index("c")
    irow0 = wid * _IDXROWS_W
    row0 = wid * _PER_W
    pltpu.sync_copy(idx_hbm.at[pl.ds(irow0, _IDXROWS_W)], idx_v)

    def fire_gathers(q, b):
        return [
            pltpu.async_copy(
                tbl_hbm.at[idx_v.at[q * _G + j]],
                rows_v.at[b].at[pl.ds(j * _IDX_W, _IDX_W)],
                gsems[b],
            )
            for j in range(_G)
        ]

    gather_cps = {}
    wb_cps = {}
    for q in range(_NCHUNK + 1):
        if q < _NCHUNK:
            b = q % _NBUF
            if q >= _NBUF:
                wb_cps.pop(q - _NBUF).wait()
            gather_cps[q] = fire_gathers(q, b)
        if q >= 1:
            qq = q - 1
            bb = qq % _NBUF
            for cp in gather_cps.pop(qq):
                cp.wait()
            wb_cps[qq] = pltpu.async_copy(
                rows_v.at[bb],
                out_hbm.at[pl.ds(row0 + qq * _CHUNK, _CHUNK)],
                wsems[bb],
            )
    for cp in wb_cps.values():
        cp.wait()


_MESH = dict(core_axis_name="c", subcore_axis_name="s")


@jax.jit
def _run(x, tables):
    tt = tables.transpose(0, 2, 1)

    edge = tables[:, _TJ * 128 :, :].reshape(_NUM_FIELDS, 8, 128)

    transpose_k = pl.kernel(
        _transpose_body,
        out_type=jax.ShapeDtypeStruct((_NUM_FIELDS * _VOCAB // 4, 128), jnp.float32),
        mesh=plsc.VectorSubcoreMesh(**_MESH),
        scratch_types=[
            pltpu.VMEM((2, _D, 128), jnp.float32),
            pltpu.VMEM((2, _D, 128), jnp.float32),
            [pltpu.SemaphoreType.DMA] * 2,
            [pltpu.SemaphoreType.DMA] * 2,
        ],
        compiler_params=pltpu.CompilerParams(
            use_tc_tiling_on_sc=True, needs_layout_passes=False
        ),
    )
    tbl = transpose_k(tt, edge).reshape(_NUM_FIELDS * _VOCAB, _D)

    offs = (jnp.arange(_NUM_FIELDS, dtype=jnp.int32) * _VOCAB)[:, None]
    idx = (x.T + offs).reshape(_TOTAL // _IDX_W, _IDX_W)
    gather_k = pl.kernel(
        _gather_body,
        out_type=jax.ShapeDtypeStruct((_TOTAL, _D), jnp.float32),
        mesh=plsc.VectorSubcoreMesh(**_MESH),
        scratch_types=[
            pltpu.VMEM((_IDXROWS_W, _IDX_W), jnp.int32),
            pltpu.VMEM((_NBUF, _CHUNK, _D), jnp.float32),
            [pltpu.SemaphoreType.DMA] * _NBUF,
            [pltpu.SemaphoreType.DMA] * _NBUF,
        ],
        compiler_params=pltpu.CompilerParams(use_tc_tiling_on_sc=False),
    )
    return gather_k(idx, tbl).reshape(_NUM_FIELDS, _BATCH, _D)


def kernel(x, tables):
    return _run(x, tables)


# speedup vs baseline: 1.7319x; 1.7319x over previous
import jax
import jax.numpy as jnp
from jax import lax
from jax.experimental import pallas as pl
from jax.experimental.pallas import tpu as pltpu
from jax.experimental.pallas import tpu_sc as plsc

_NUM_FIELDS = 26
_VOCAB = 100000
_D = 32
_BATCH = 16384
_NC, _NS = 2, 16
_NW = _NC * _NS

_TJ = 781
_PAIRS = _NUM_FIELDS * _TJ
_SPW = 636

_TOTAL = _NUM_FIELDS * _BATCH
_PER_W = _TOTAL // _NW
_IDX_W = 128
_IDXROWS_W = _PER_W // _IDX_W
_G = 8
_CHUNK = _G * _IDX_W
_NCHUNK = _IDXROWS_W // _G
_NBUF = 3


def _pair_of(wid, s):
    p = s * _NW + wid
    valid = p < _PAIRS
    pc = jnp.where(valid, p, 0)
    return valid, pc // _TJ, pc % _TJ


def _transpose_body(tt_hbm, edge_hbm, tbl_hbm, tin, tout, isems, osems):
    wid = lax.axis_index("s") * _NC + lax.axis_index("c")
    nb = 2
    iota = lax.iota(jnp.int32, 16)
    dvec0 = iota
    dvec1 = iota + 16

    def fire_read(s, b):
        valid, f, tj = _pair_of(wid, s)

        @pl.when(valid)
        def _():
            for t in range(4):
                pltpu.async_copy(
                    tt_hbm.at[f, pl.ds(t * 8, 8), pl.ds(tj * 128, 128)],
                    tin.at[b].at[pl.ds(t * 8, 8)],
                    isems[b],
                )

    def wait_read(b):
        for t in range(4):
            pltpu.make_async_copy(
                tt_hbm.at[0, pl.ds(0, 8), pl.ds(0, 128)],
                tin.at[b].at[pl.ds(t * 8, 8)],
                isems[b],
            ).wait()

    def wait_write(b):
        pltpu.make_async_copy(
            tout.at[b], tbl_hbm.at[pl.ds(0, _D)], osems[b]
        ).wait()

    def step(s, b, first):
        valid, f, tj = _pair_of(wid, s)

        @pl.when(valid)
        def _():
            wait_read(b)
            if not first:
                wait_write(b)
            bvec = jnp.full((16,), b, jnp.int32)
            for r in range(0, _D, 2):
                vals = []
                for rr in (r, r + 1):
                    for m in range(8):
                        dvec = dvec0 if m % 2 == 0 else dvec1
                        cvec = jnp.full((16,), rr * 4 + m // 2, jnp.int32)
                        vals.append(plsc.load_gather(tin, [bvec, dvec, cvec]))
                for i, rr in enumerate((r, r + 1)):
                    for m in range(8):
                        tout[b, rr, pl.ds(m * 16, 16)] = vals[i * 8 + m]
            k0 = (f * (_VOCAB // 32) + tj * 4) * 8
            pltpu.async_copy(tout.at[b], tbl_hbm.at[pl.ds(k0, _D)], osems[b])

        fire_read(s + nb, b)

    fire_read(0, 0)
    fire_read(1, 1)
    step(0, 0, True)
    step(1, 1, True)

    def body(t2, carry):
        s0 = nb + 2 * t2
        step(s0, 0, False)
        step(s0 + 1, 1, False)
        return carry

    lax.fori_loop(0, (_SPW - nb) // 2, body, 0)

    for q in range(_SPW - nb, _SPW):
        valid, f, tj = _pair_of(wid, q)

        @pl.when(valid)
        def _():
            wait_write(q % nb)

    @pl.when(wid < _NUM_FIELDS)
    def _():
        f = wid
        pltpu.sync_copy(edge_hbm.at[f], tin.at[0, pl.ds(0, 8), pl.ds(0, 128)])
        k0 = f * (_VOCAB // 4) + (_TJ * 128) // 4
        pltpu.sync_copy(
            tin.at[0, pl.ds(0, 8), pl.ds(0, 128)], tbl_hbm.at[pl.ds(k0, 8)]
        )


def _gather_body(idx_hbm, tbl_hbm, out_hbm, idx_v, rows_v, gsems, wsems):
    wid = lax.axis_index("s") * _NC + lax.axis_index("c")
    irow0 = wid * _IDXROWS_W
    row0 = wid * _PER_W
    pltpu.sync_copy(idx_hbm.at[pl.ds(irow0, _IDXROWS_W)], idx_v)

    def fire_gathers(q, b):
        return [
            pltpu.async_copy(
                tbl_hbm.at[idx_v.at[q * _G + j]],
                rows_v.at[b].at[pl.ds(j * _IDX_W, _IDX_W)],
                gsems[b],
            )
            for j in range(_G)
        ]

    gather_cps = {}
    wb_cps = {}
    for q in range(_NCHUNK + 1):
        if q < _NCHUNK:
            b = q % _NBUF
            if q >= _NBUF:
                wb_cps.pop(q - _NBUF).wait()
            gather_cps[q] = fire_gathers(q, b)
        if q >= 1:
            qq = q - 1
            bb = qq % _NBUF
            for cp in gather_cps.pop(qq):
                cp.wait()
            wb_cps[qq] = pltpu.async_copy(
                rows_v.at[bb],
                out_hbm.at[pl.ds(row0 + qq * _CHUNK, _CHUNK)],
                wsems[bb],
            )
    for cp in wb_cps.values():
        cp.wait()


_MESH = dict(core_axis_name="c", subcore_axis_name="s")


@jax.jit
def _run(x, tables):
    tt = tables.transpose(0, 2, 1)

    edge = tables[:, _TJ * 128 :, :].reshape(_NUM_FIELDS, 8, 128)

    transpose_k = pl.kernel(
        _transpose_body,
        out_type=jax.ShapeDtypeStruct((_NUM_FIELDS * _VOCAB // 4, 128), jnp.float32),
        mesh=plsc.VectorSubcoreMesh(**_MESH),
        scratch_types=[
            pltpu.VMEM((2, _D, 128), jnp.float32),
            pltpu.VMEM((2, _D, 128), jnp.float32),
            [pltpu.SemaphoreType.DMA] * 2,
            [pltpu.SemaphoreType.DMA] * 2,
        ],
        compiler_params=pltpu.CompilerParams(
            use_tc_tiling_on_sc=True, needs_layout_passes=False
        ),
    )
    tbl = transpose_k(tt, edge).reshape(_NUM_FIELDS * _VOCAB, _D)

    offs = (jnp.arange(_NUM_FIELDS, dtype=jnp.int32) * _VOCAB)[:, None]
    idx = (x.T + offs).reshape(_TOTAL // _IDX_W, _IDX_W)
    gather_k = pl.kernel(
        _gather_body,
        out_type=jax.ShapeDtypeStruct((_TOTAL, _D), jnp.float32),
        mesh=plsc.VectorSubcoreMesh(**_MESH),
        scratch_types=[
            pltpu.VMEM((_IDXROWS_W, _IDX_W), jnp.int32),
            pltpu.VMEM((_NBUF, _CHUNK, _D), jnp.float32),
            [pltpu.SemaphoreType.DMA] * _NBUF,
            [pltpu.SemaphoreType.DMA] * _NBUF,
        ],
        compiler_params=pltpu.CompilerParams(use_tc_tiling_on_sc=False),
    )
    return gather_k(idx, tbl).reshape(_NUM_FIELDS, _BATCH, _D)


def kernel(x, tables):
    return _run(x, tables)


# speedup vs baseline: 2.1946x; 1.2671x over previous
import jax
import jax.numpy as jnp
from jax import lax
from jax.experimental import pallas as pl
from jax.experimental.pallas import tpu as pltpu
from jax.experimental.pallas import tpu_sc as plsc

_NUM_FIELDS = 26
_VOCAB = 100000
_D = 32
_BATCH = 16384
_NC, _NS = 2, 16
_NW = _NC * _NS
_TOTAL = _NUM_FIELDS * _BATCH
_PER_W = _TOTAL // _NW
_IDX_W = 128
_IDXROWS_W = _PER_W // _IDX_W
_G = 8
_CHUNK = _G * _IDX_W
_NCHUNK = _IDXROWS_W // _G
_NBUF = 3


def _body(idx_hbm, table_hbm, out_hbm, idx_v, rows_v, gsems, osems):
    wid = lax.axis_index("s") * _NC + lax.axis_index("c")
    irow0 = wid * _IDXROWS_W
    row0 = wid * _PER_W
    pltpu.sync_copy(idx_hbm.at[pl.ds(irow0, _IDXROWS_W)], idx_v)

    def fire_gathers(q, b):
        return [
            pltpu.async_copy(
                table_hbm.at[idx_v.at[q * _G + j]],
                rows_v.at[b].at[pl.ds(j * _IDX_W, _IDX_W)],
                gsems[b],
            )
            for j in range(_G)
        ]

    gather_cps = {}
    wb_cps = {}
    for q in range(_NCHUNK + 1):
        if q < _NCHUNK:
            b = q % _NBUF
            if q >= _NBUF:
                wb_cps.pop(q - _NBUF).wait()
            gather_cps[q] = fire_gathers(q, b)
        if q >= 1:
            qq = q - 1
            bb = qq % _NBUF
            for cp in gather_cps.pop(qq):
                cp.wait()
            wb_cps[qq] = pltpu.async_copy(
                rows_v.at[bb],
                out_hbm.at[pl.ds(row0 + qq * _CHUNK, _CHUNK), pl.ds(0, _D)],
                osems[bb],
            )
    for cp in wb_cps.values():
        cp.wait()


@jax.jit
def _run(idx, table_flat):
    k = pl.kernel(
        _body,
        out_type=jax.ShapeDtypeStruct((_TOTAL, 128), jnp.float32),
        mesh=plsc.VectorSubcoreMesh(core_axis_name="c", subcore_axis_name="s"),
        scratch_types=[
            pltpu.VMEM((_IDXROWS_W, _IDX_W), jnp.int32),
            pltpu.VMEM((_NBUF, _CHUNK, _D), jnp.float32),
            [pltpu.SemaphoreType.DMA] * _NBUF,
            [pltpu.SemaphoreType.DMA] * _NBUF,
        ],
        compiler_params=pltpu.CompilerParams(use_tc_tiling_on_sc=False),
    )
    return k(idx, table_flat)


def kernel(x, tables):
    offs = (jnp.arange(_NUM_FIELDS, dtype=jnp.int32) * _VOCAB)[:, None]
    idx = (x.T + offs).reshape(_TOTAL // _IDX_W, _IDX_W)
    out = _run(idx, tables.reshape(_NUM_FIELDS * _VOCAB, _D))
    return out.reshape(_NUM_FIELDS, _BATCH, 128)[:, :, :_D]
